# Initial kernel scaffold; baseline (speedup 1.0000x reference)
import jax
import jax.numpy as jnp
from jax.experimental import pallas as pl


def kernel(h, r, t, ent_table, rel_table):
    raise NotImplementedError("write your pallas kernel here")


# speedup vs baseline: 2.2783x; 2.2783x over previous
import functools

import jax
import jax.numpy as jnp
import numpy as np
from jax import lax
from jax.experimental import pallas as pl
from jax.experimental.pallas import tpu as pltpu
from jax.experimental.pallas import tpu_sc as plsc

B = 16384
D = 128
HALF = D // 2
L = 16
NC, NS = 2, 16
NW = NC * NS
BPW = B // NW
C = 128
NCHUNK = BPW // C

_mesh = plsc.VectorSubcoreMesh(core_axis_name="c", subcore_axis_name="s")


_PERM_DNUMS = lax.GatherDimensionNumbers(
    offset_dims=(), collapsed_slice_dims=(0,), start_index_map=(0,))


def _perm(v, idx):
    return lax.gather(v, idx[:, None], _PERM_DNUMS, (1,),
                      mode=lax.GatherScatterMode.PROMISE_IN_BOUNDS)


@functools.partial(
    pl.kernel,
    mesh=_mesh,
    out_type=jax.ShapeDtypeStruct((B,), jnp.float32),
    scratch_types=[
        pltpu.VMEM((BPW,), jnp.int32),
        pltpu.VMEM((BPW,), jnp.int32),
        pltpu.VMEM((BPW,), jnp.int32),
        pltpu.VMEM((C, D), jnp.float32),
        pltpu.VMEM((C, D), jnp.float32),
        pltpu.VMEM((C, D), jnp.float32),
        pltpu.VMEM((BPW,), jnp.float32),
        pltpu.SemaphoreType.DMA,
    ],
)
def _complex_score_sc(h_hbm, r_hbm, t_hbm, ent_hbm, rel_hbm, out_hbm,
                      idx_h, idx_r, idx_t, rows_h, rows_r, rows_t,
                      outb, sem):
    wid = lax.axis_index("s") * NC + lax.axis_index("c")
    base = wid * BPW

    lane = lax.iota(jnp.int32, L)
    inv = ((lane & 1) << 3) | ((lane & 2) << 1) | ((lane & 4) >> 1) | ((lane & 8) >> 3)
    fold_idx = {blk: lane ^ blk for blk in (8, 4, 2, 1)}
    merge_mask = {blk: (lane & blk) == 0 for blk in (8, 4, 2, 1)}

    pltpu.sync_copy(h_hbm.at[pl.ds(base, BPW)], idx_h)
    pltpu.sync_copy(r_hbm.at[pl.ds(base, BPW)], idx_r)
    pltpu.sync_copy(t_hbm.at[pl.ds(base, BPW)], idx_t)

    for ck in range(NCHUNK):
        co = ck * C
        cp_h = pltpu.async_copy(ent_hbm.at[idx_h.at[pl.ds(co, C)]], rows_h, sem)
        cp_r = pltpu.async_copy(rel_hbm.at[idx_r.at[pl.ds(co, C)]], rows_r, sem)
        cp_t = pltpu.async_copy(ent_hbm.at[idx_t.at[pl.ds(co, C)]], rows_t, sem)
        cp_h.wait()
        cp_r.wait()
        cp_t.wait()

        def group_body(g, _, co=co):
            ro = g * L
            vecs = []
            for i in range(L):
                acc = jnp.zeros((L,), jnp.float32)
                for j in range(HALF // L):
                    hr = rows_h[ro + i, pl.ds(j * L, L)]
                    hi = rows_h[ro + i, pl.ds(HALF + j * L, L)]
                    rr = rows_r[ro + i, pl.ds(j * L, L)]
                    ri = rows_r[ro + i, pl.ds(HALF + j * L, L)]
                    tr = rows_t[ro + i, pl.ds(j * L, L)]
                    ti = rows_t[ro + i, pl.ds(HALF + j * L, L)]
                    a = hr * rr - hi * ri
                    b = hr * ri + hi * rr
                    acc = acc + a * tr + b * ti
                vecs.append(acc)
            block = L // 2
            while len(vecs) > 1:
                folded = [v + _perm(v, fold_idx[block]) for v in vecs]
                vecs = [jnp.where(merge_mask[block], folded[p], folded[p + 1])
                        for p in range(0, len(folded), 2)]
                block //= 2
            res = _perm(vecs[0], inv)
            outb[pl.ds(co + ro, L)] = res
            return 0

        lax.fori_loop(0, C // L, group_body, 0)

    pltpu.sync_copy(outb, out_hbm.at[pl.ds(base, BPW)])


def kernel(h, r, t, ent_table, rel_table):
    score = _complex_score_sc(h, r, t, ent_table, rel_table)
    return score[:, None]


# speedup vs baseline: 2.6201x; 1.1500x over previous
import functools

import jax
import jax.numpy as jnp
import numpy as np
from jax import lax
from jax.experimental import pallas as pl
from jax.experimental.pallas import tpu as pltpu
from jax.experimental.pallas import tpu_sc as plsc

B = 16384
D = 128
HALF = D // 2
L = 16
NC, NS = 2, 16
NW = NC * NS
BPW = B // NW
C = 128
NCHUNK = BPW // C

_mesh = plsc.VectorSubcoreMesh(core_axis_name="c", subcore_axis_name="s")


_PERM_DNUMS = lax.GatherDimensionNumbers(
    offset_dims=(), collapsed_slice_dims=(0,), start_index_map=(0,))


def _perm(v, idx):
    return lax.gather(v, idx[:, None], _PERM_DNUMS, (1,),
                      mode=lax.GatherScatterMode.PROMISE_IN_BOUNDS)


@functools.partial(
    pl.kernel,
    mesh=_mesh,
    out_type=jax.ShapeDtypeStruct((B,), jnp.float32),
    scratch_types=[
        pltpu.VMEM((BPW,), jnp.int32),
        pltpu.VMEM((BPW,), jnp.int32),
        pltpu.VMEM((BPW,), jnp.int32),
        pltpu.VMEM((2, C, D), jnp.float32),
        pltpu.VMEM((2, C, D), jnp.float32),
        pltpu.VMEM((2, C, D), jnp.float32),
        pltpu.VMEM((BPW,), jnp.float32),
        pltpu.SemaphoreType.DMA,
        pltpu.SemaphoreType.DMA,
    ],
)
def _complex_score_sc(h_hbm, r_hbm, t_hbm, ent_hbm, rel_hbm, out_hbm,
                      idx_h, idx_r, idx_t, rows_h, rows_r, rows_t,
                      outb, sem0, sem1):
    wid = lax.axis_index("s") * NC + lax.axis_index("c")
    base = wid * BPW

    lane = lax.iota(jnp.int32, L)
    inv = ((lane & 1) << 3) | ((lane & 2) << 1) | ((lane & 4) >> 1) | ((lane & 8) >> 3)
    fold_idx = {blk: lane ^ blk for blk in (8, 4, 2, 1)}
    merge_mask = {blk: (lane & blk) == 0 for blk in (8, 4, 2, 1)}

    pltpu.sync_copy(h_hbm.at[pl.ds(base, BPW)], idx_h)
    pltpu.sync_copy(r_hbm.at[pl.ds(base, BPW)], idx_r)
    pltpu.sync_copy(t_hbm.at[pl.ds(base, BPW)], idx_t)

    def issue(ck):
        p = ck % 2
        co = ck * C
        s = sem0 if p == 0 else sem1
        return (
            pltpu.async_copy(ent_hbm.at[idx_h.at[pl.ds(co, C)]], rows_h.at[p], s),
            pltpu.async_copy(rel_hbm.at[idx_r.at[pl.ds(co, C)]], rows_r.at[p], s),
            pltpu.async_copy(ent_hbm.at[idx_t.at[pl.ds(co, C)]], rows_t.at[p], s),
        )

    pending = issue(0)
    for ck in range(NCHUNK):
        p = ck % 2
        co = ck * C
        nxt = issue(ck + 1) if ck + 1 < NCHUNK else None
        for cp in pending:
            cp.wait()
        pending = nxt

        def group_body(g, _, co=co, p=p):
            ro = g * L
            vecs = []
            for i in range(L):
                acc = jnp.zeros((L,), jnp.float32)
                for j in range(HALF // L):
                    hr = rows_h[p, ro + i, pl.ds(j * L, L)]
                    hi = rows_h[p, ro + i, pl.ds(HALF + j * L, L)]
                    rr = rows_r[p, ro + i, pl.ds(j * L, L)]
                    ri = rows_r[p, ro + i, pl.ds(HALF + j * L, L)]
                    tr = rows_t[p, ro + i, pl.ds(j * L, L)]
                    ti = rows_t[p, ro + i, pl.ds(HALF + j * L, L)]
                    a = hr * rr - hi * ri
                    b = hr * ri + hi * rr
                    acc = acc + a * tr + b * ti
                vecs.append(acc)
            block = L // 2
            while len(vecs) > 1:
                folded = [v + _perm(v, fold_idx[block]) for v in vecs]
                vecs = [jnp.where(merge_mask[block], folded[p], folded[p + 1])
                        for p in range(0, len(folded), 2)]
                block //= 2
            res = _perm(vecs[0], inv)
            outb[pl.ds(co + ro, L)] = res
            return 0

        lax.fori_loop(0, C // L, group_body, 0)

    pltpu.sync_copy(outb, out_hbm.at[pl.ds(base, BPW)])


def kernel(h, r, t, ent_table, rel_table):
    score = _complex_score_sc(h, r, t, ent_table, rel_table)
    return score[:, None]


# speedup vs baseline: 3.1604x; 1.2062x over previous
import functools

import jax
import jax.numpy as jnp
import numpy as np
from jax import lax
from jax.experimental import pallas as pl
from jax.experimental.pallas import tpu as pltpu
from jax.experimental.pallas import tpu_sc as plsc

B = 16384
D = 128
HALF = D // 2
L = 16
NC, NS = 2, 16
NW = NC * NS
BPW = B // NW
C = 128
NCHUNK = BPW // C

_mesh = plsc.VectorSubcoreMesh(core_axis_name="c", subcore_axis_name="s")


_PERM_DNUMS = lax.GatherDimensionNumbers(
    offset_dims=(), collapsed_slice_dims=(0,), start_index_map=(0,))


def _perm(v, idx):
    return lax.gather(v, idx[:, None], _PERM_DNUMS, (1,),
                      mode=lax.GatherScatterMode.PROMISE_IN_BOUNDS)


@functools.partial(
    pl.kernel,
    mesh=_mesh,
    out_type=jax.ShapeDtypeStruct((B,), jnp.float32),
    scratch_types=[
        pltpu.VMEM((BPW,), jnp.int32),
        pltpu.VMEM((BPW,), jnp.int32),
        pltpu.VMEM((BPW,), jnp.int32),
        pltpu.VMEM((2, C, D), jnp.float32),
        pltpu.VMEM((2, C, D), jnp.float32),
        pltpu.VMEM((2, C, D), jnp.float32),
        pltpu.VMEM((C // 2 * L,), jnp.float32),
        pltpu.VMEM((BPW,), jnp.float32),
        pltpu.SemaphoreType.DMA,
        pltpu.SemaphoreType.DMA,
    ],
)
def _complex_score_sc(h_hbm, r_hbm, t_hbm, ent_hbm, rel_hbm, out_hbm,
                      idx_h, idx_r, idx_t, rows_h, rows_r, rows_t,
                      pairbuf, outb, sem0, sem1):
    wid = lax.axis_index("s") * NC + lax.axis_index("c")
    base = wid * BPW

    lane = lax.iota(jnp.int32, L)
    inv = ((lane & 1) << 3) | ((lane & 2) << 1) | ((lane & 4) >> 1) | ((lane & 8) >> 3)
    fold_idx = {blk: lane ^ blk for blk in (8, 4, 2, 1)}
    merge_mask = {blk: (lane & blk) == 0 for blk in (8, 4, 2, 1)}

    pltpu.sync_copy(h_hbm.at[pl.ds(base, BPW)], idx_h)
    pltpu.sync_copy(r_hbm.at[pl.ds(base, BPW)], idx_r)
    pltpu.sync_copy(t_hbm.at[pl.ds(base, BPW)], idx_t)

    def issue(ck):
        p = ck % 2
        co = ck * C
        s = sem0 if p == 0 else sem1
        return (
            pltpu.async_copy(ent_hbm.at[idx_h.at[pl.ds(co, C)]], rows_h.at[p], s),
            pltpu.async_copy(rel_hbm.at[idx_r.at[pl.ds(co, C)]], rows_r.at[p], s),
            pltpu.async_copy(ent_hbm.at[idx_t.at[pl.ds(co, C)]], rows_t.at[p], s),
        )

    pending = issue(0)
    for ck in range(NCHUNK):
        p = ck % 2
        co = ck * C
        nxt = issue(ck + 1) if ck + 1 < NCHUNK else None
        for cp in pending:
            cp.wait()
        pending = nxt

        def row_acc(ro, p):
            acc = jnp.zeros((L,), jnp.float32)
            for j in range(HALF // L):
                hr = rows_h[p, ro, pl.ds(j * L, L)]
                hi = rows_h[p, ro, pl.ds(HALF + j * L, L)]
                rr = rows_r[p, ro, pl.ds(j * L, L)]
                ri = rows_r[p, ro, pl.ds(HALF + j * L, L)]
                tr = rows_t[p, ro, pl.ds(j * L, L)]
                ti = rows_t[p, ro, pl.ds(HALF + j * L, L)]
                a = hr * rr - hi * ri
                b = hr * ri + hi * rr
                acc = acc + a * tr + b * ti
            return acc

        def pair_body(q, _, p=p):
            a_v = row_acc(2 * q, p)
            b_v = row_acc(2 * q + 1, p)
            fa = a_v + _perm(a_v, fold_idx[L // 2])
            fb = b_v + _perm(b_v, fold_idx[L // 2])
            pairbuf[pl.ds(q * L, L)] = jnp.where(merge_mask[L // 2], fa, fb)
            return 0

        def group_body(g, _, co=co):
            stack = []
            for q in range(L // 2):
                stack.append((1, pairbuf[pl.ds((g * (L // 2) + q) * L, L)]))
                while len(stack) >= 2 and stack[-1][0] == stack[-2][0]:
                    lvl, b_v = stack.pop()
                    _, a_v = stack.pop()
                    blk = (L // 2) >> lvl
                    fa = a_v + _perm(a_v, fold_idx[blk])
                    fb = b_v + _perm(b_v, fold_idx[blk])
                    stack.append((lvl + 1, jnp.where(merge_mask[blk], fa, fb)))
            res = _perm(stack[0][1], inv)
            outb[pl.ds(co + g * L, L)] = res
            return 0

        lax.fori_loop(0, C // 2, pair_body, 0)
        lax.fori_loop(0, C // L, group_body, 0)

    pltpu.sync_copy(outb, out_hbm.at[pl.ds(base, BPW)])


def kernel(h, r, t, ent_table, rel_table):
    score = _complex_score_sc(h, r, t, ent_table, rel_table)
    return score[:, None]
